# baseline (speedup 1.0000x reference)
import jax
import jax.numpy as jnp
from jax import lax
from jax.experimental import pallas as pl
from jax.experimental.pallas import tpu as pltpu
from jax.experimental.pallas import tpu_sc as plsc

V = 10000
E = 320000
D = 128

NC = 2
NS = 16
VH = V // NC
VPAD = 5008
BLK = 128
NB = E // BLK
NBUF = 3
V_TILE = 312
V_LAST = VH - (NS - 1) * V_TILE


def _sc_body(h_hbm, ids_hbm, n0_hbm, out_hbm, *scratch):
    rows = scratch[0:NBUF]
    idxs = scratch[NBUF:2 * NBUF]
    gsems = scratch[2 * NBUF:3 * NBUF]
    ssems = scratch[3 * NBUF:4 * NBUF]
    n0_v = scratch[4 * NBUF]
    acc = scratch[4 * NBUF + 1]

    c = lax.axis_index("c")
    s = lax.axis_index("s")

    pltpu.sync_copy(n0_hbm, n0_v)
    n0 = n0_v[...][0]

    z16 = jnp.zeros((16,), jnp.float32)

    def zrow(r, carry):
        for g in range(D // 16):
            rows[0][r, pl.ds(g * 16, 16)] = z16
        return carry

    lax.fori_loop(0, BLK, zrow, 0)
    for r0 in range(0, 313, BLK):
        n = min(BLK, 313 - r0)
        pltpu.sync_copy(rows[0].at[pl.ds(0, n)],
                        acc.at[pl.ds(s * 313 + r0, n)])

    plsc.subcore_barrier()

    blo = jnp.where(c == 0, 0, n0 // BLK)
    bhi = jnp.where(c == 0, (n0 + BLK - 1) // BLK, NB)
    nbc = bhi - blo
    lo = blo + nbc * s // NS
    nt = blo + nbc * (s + 1) // NS - lo

    vbase = c * VH

    def start_gather(ch, k):
        pltpu.async_copy(h_hbm.at[pl.ds((lo + ch) * BLK, BLK)], rows[k],
                         gsems[k])
        pltpu.async_copy(ids_hbm.at[pl.ds((lo + ch) * BLK, BLK)], idxs[k],
                         gsems[k])

    def wait_gather(k):
        pltpu.make_async_copy(h_hbm.at[pl.ds(0, BLK)], rows[k],
                              gsems[k]).wait()
        pltpu.make_async_copy(ids_hbm.at[pl.ds(0, BLK)], idxs[k],
                              gsems[k]).wait()

    def remap(k):
        for g in range(BLK // 16):
            sl = pl.ds(g * 16, 16)
            vid = idxs[k][sl] - vbase
            ok = (vid >= 0) & (vid < VH)
            idxs[k][sl] = jnp.where(ok, vid, VH)

    def start_scatter(k):
        pltpu.async_copy(rows[k], acc.at[idxs[k]], ssems[k], add=True)

    def wait_scatter(k):
        pltpu.make_async_copy(rows[k], acc.at[idxs[k]], ssems[k]).wait()

    for k in range(NBUF):
        @pl.when(k < nt)
        def _():
            start_gather(k, k)

    def round_(p, carry):
        for k in range(NBUF):
            ch = NBUF * p + k

            @pl.when(ch < nt)
            def _():
                wait_gather(k)
                remap(k)
                start_scatter(k)

                @pl.when(ch + NBUF < nt)
                def _():
                    wait_scatter(k)
                    start_gather(ch + NBUF, k)
        return carry

    lax.fori_loop(0, (nt + NBUF - 1) // NBUF, round_, 0)
    for k in range(NBUF):
        @pl.when(k < nt)
        def _():
            wait_scatter(k)

    plsc.subcore_barrier()

    pltpu.sync_copy(acc.at[pl.ds(s * V_TILE, V_TILE)],
                    out_hbm.at[pl.ds(vbase + s * V_TILE, V_TILE)])

    @pl.when(s == NS - 1)
    def _():
        tail = (NS - 1) * V_TILE + V_TILE
        pltpu.sync_copy(acc.at[pl.ds(tail, V_LAST - V_TILE)],
                        out_hbm.at[pl.ds(vbase + tail, V_LAST - V_TILE)])


@jax.jit
def _segment_sum_sc(H, ids, n0):
    mesh = plsc.VectorSubcoreMesh(core_axis_name="c", subcore_axis_name="s")
    return pl.kernel(
        _sc_body,
        out_type=jax.ShapeDtypeStruct((V, D), jnp.float32),
        mesh=mesh,
        scratch_types=(
            [pltpu.VMEM((BLK, D), jnp.float32) for _ in range(NBUF)]
            + [pltpu.VMEM((BLK,), jnp.int32) for _ in range(NBUF)]
            + [pltpu.SemaphoreType.DMA for _ in range(2 * NBUF)]
            + [pltpu.VMEM((16,), jnp.int32)]
            + [pltpu.VMEM_SHARED((VPAD, D), jnp.float32)]
        ),
    )(H, ids, n0)


def kernel(H, X_node):
    ids = X_node.astype(jnp.int32)
    n0 = jnp.full((16,), jnp.searchsorted(ids, VH), jnp.int32)
    return _segment_sum_sc(H, ids, n0)


# speedup vs baseline: 1.1137x; 1.1137x over previous
import jax
import jax.numpy as jnp
from jax import lax
from jax.experimental import pallas as pl
from jax.experimental.pallas import tpu as pltpu
from jax.experimental.pallas import tpu_sc as plsc

V = 10000
E = 320000
D = 128

NC = 2
NS = 16
NW = NC * NS
BLK = 128
NB = E // BLK
NBUF = 3
WBLK = NB // NW
NTRI = WBLK // NBUF
TAIL = NB - NW * WBLK
V_TILE = 624
V_LAST = V - (NS - 1) * V_TILE


def _acc_slab(s):
    return pl.ds(s * V_TILE, V_TILE)


def _acc_tail():
    return pl.ds((NS - 1) * V_TILE + V_TILE, V_LAST - V_TILE)


def _sc_body(h_hbm, ids_hbm, part_hbm, *scratch):
    rows = scratch[0:NBUF]
    idxs = scratch[NBUF:2 * NBUF]
    gsems = scratch[2 * NBUF:3 * NBUF]
    ssems = scratch[3 * NBUF:4 * NBUF]
    acc = scratch[4 * NBUF]

    c = lax.axis_index("c")
    s = lax.axis_index("s")
    wid = c * NS + s

    z16 = jnp.zeros((16,), jnp.float32)

    def zrow(r, carry):
        for g in range(D // 16):
            rows[0][r, pl.ds(g * 16, 16)] = z16
        return carry

    lax.fori_loop(0, BLK, zrow, 0)
    for r0 in range(0, V_TILE, BLK):
        n = min(BLK, V_TILE - r0)
        pltpu.sync_copy(rows[0].at[pl.ds(0, n)],
                        acc.at[pl.ds(s * V_TILE + r0, n)])

    @pl.when(s == NS - 1)
    def _():
        pltpu.sync_copy(rows[0].at[pl.ds(0, V_LAST - V_TILE)],
                        acc.at[_acc_tail()])

    plsc.subcore_barrier()

    b0 = wid * WBLK

    def start_gather(ch, k):
        pltpu.async_copy(h_hbm.at[pl.ds((b0 + ch) * BLK, BLK)], rows[k],
                         gsems[k])
        pltpu.async_copy(ids_hbm.at[pl.ds((b0 + ch) * BLK, BLK)], idxs[k],
                         gsems[k])

    def wait_gather(k):
        pltpu.make_async_copy(h_hbm.at[pl.ds(0, BLK)], rows[k],
                              gsems[k]).wait()
        pltpu.make_async_copy(ids_hbm.at[pl.ds(0, BLK)], idxs[k],
                              gsems[k]).wait()

    def start_scatter(k):
        pltpu.async_copy(rows[k], acc.at[idxs[k]], ssems[k], add=True)

    def wait_scatter(k):
        pltpu.make_async_copy(rows[k], acc.at[idxs[k]], ssems[k]).wait()

    for k in range(NBUF):
        start_gather(k, k)

    def triple(p, carry):
        for k in range(NBUF):
            ch = NBUF * p + k
            wait_gather(k)
            start_scatter(k)

            @pl.when(ch + NBUF < WBLK)
            def _():
                wait_scatter(k)
                start_gather(ch + NBUF, k)
        return carry

    lax.fori_loop(0, NTRI, triple, 0)
    for k in range(NBUF):
        wait_scatter(k)

    @pl.when(wid < TAIL)
    def _():
        tb = NW * WBLK + wid
        pltpu.sync_copy(h_hbm.at[pl.ds(tb * BLK, BLK)], rows[0])
        pltpu.sync_copy(ids_hbm.at[pl.ds(tb * BLK, BLK)], idxs[0])
        pltpu.sync_copy(rows[0], acc.at[idxs[0]], add=True)

    plsc.subcore_barrier()

    pltpu.sync_copy(acc.at[_acc_slab(s)], part_hbm.at[c, _acc_slab(s)])

    @pl.when(s == NS - 1)
    def _():
        pltpu.sync_copy(acc.at[_acc_tail()], part_hbm.at[c, _acc_tail()])


@jax.jit
def _segment_sum_sc(H, ids):
    mesh = plsc.VectorSubcoreMesh(core_axis_name="c", subcore_axis_name="s")
    return pl.kernel(
        _sc_body,
        out_type=jax.ShapeDtypeStruct((NC, V, D), jnp.float32),
        mesh=mesh,
        scratch_types=(
            [pltpu.VMEM((BLK, D), jnp.float32) for _ in range(NBUF)]
            + [pltpu.VMEM((BLK,), jnp.int32) for _ in range(NBUF)]
            + [pltpu.SemaphoreType.DMA for _ in range(2 * NBUF)]
            + [pltpu.VMEM_SHARED((V, D), jnp.float32)]
        ),
    )(H, ids)


def _tc_add_body(p0, p1, o):
    o[...] = p0[...] + p1[...]


@jax.jit
def _combine(part):
    blk = 1000
    grid = V // blk
    spec = pl.BlockSpec((blk, D), lambda i: (i, 0))
    return pl.pallas_call(
        _tc_add_body,
        grid=(grid,),
        in_specs=[spec, spec],
        out_specs=spec,
        out_shape=jax.ShapeDtypeStruct((V, D), jnp.float32),
    )(part[0], part[1])


def kernel(H, X_node):
    ids = X_node.astype(jnp.int32)
    part = _segment_sum_sc(H, ids)
    return _combine(part)
